# baseline (speedup 1.0000x reference)
import functools

import jax
import jax.numpy as jnp
from jax import lax
from jax.experimental import pallas as pl
from jax.experimental.pallas import tpu as pltpu
from jax.experimental.pallas import tpu_sc as plsc

NC = 2
NS = 16
L = 16
NW = NC * NS
CH = 128


def _make_sc_kernel(B: int, D: int):
    bpw = B // NW
    nch = bpw // CH
    gpc = CH // L
    mesh = plsc.VectorSubcoreMesh(core_axis_name="c", subcore_axis_name="s")

    @functools.partial(
        pl.kernel,
        out_type=jax.ShapeDtypeStruct((B,), jnp.float32),
        mesh=mesh,
        compiler_params=pltpu.CompilerParams(
            needs_layout_passes=False, use_tc_tiling_on_sc=False),
        scratch_types=[
            pltpu.VMEM((nch, CH), jnp.int32),
            pltpu.VMEM((nch, CH), jnp.int32),
            pltpu.VMEM((nch, CH, D), jnp.float32),
            pltpu.VMEM((nch, CH, D), jnp.float32),
            pltpu.VMEM((bpw,), jnp.float32),
            pltpu.VMEM((L,), jnp.float32),
            pltpu.SemaphoreType.DMA((nch + 2,)),
        ],
    )
    def sc_kernel(src_hbm, tgt_hbm, table_hbm, w_hbm, b_hbm, out_hbm,
                  idx_s, idx_t, rows_s, rows_t, out_v, wb_sm, sems):
        wid = lax.axis_index("s") * NC + lax.axis_index("c")
        base = wid * bpw

        c_is = pltpu.async_copy(src_hbm.at[pl.ds(wid * nch, nch)], idx_s,
                                sems.at[nch])
        c_it = pltpu.async_copy(tgt_hbm.at[pl.ds(wid * nch, nch)], idx_t,
                                sems.at[nch + 1])
        c_is.wait()
        c_it.wait()
        copies = []
        for j in range(nch):
            copies.append((
                pltpu.async_copy(table_hbm.at[idx_s.at[j]], rows_s.at[j], sems.at[j]),
                pltpu.async_copy(table_hbm.at[idx_t.at[j]], rows_t.at[j], sems.at[j]),
            ))
        c_w = pltpu.async_copy(w_hbm, wb_sm.at[pl.ds(0, 1)], sems.at[nch])
        c_b = pltpu.async_copy(b_hbm, wb_sm.at[pl.ds(8, 1)], sems.at[nch + 1])
        c_w.wait()
        c_b.wait()

        wb_vec = wb_sm[...]
        w_vec = jnp.full((L,), wb_vec[0], jnp.float32)
        b_vec = jnp.full((L,), wb_vec[8], jnp.float32)
        lane = lax.iota(jnp.int32, L)

        for j in range(nch):
            cs, ct = copies[j]
            cs.wait()
            ct.wait()

            def group(g, _, j=j):
                rows = g * L + lane
                acc = jnp.zeros((L,), jnp.float32)
                for d in range(D):
                    dcol = jnp.full((L,), d, jnp.int32)
                    sv = plsc.load_gather(rows_s.at[j], [rows, dcol])
                    tv = plsc.load_gather(rows_t.at[j], [rows, dcol])
                    acc = acc + sv * tv
                out_v[pl.ds(j * CH + g * L, L)] = acc * w_vec + b_vec
                return _

            lax.fori_loop(0, gpc, group, 0)

        pltpu.sync_copy(out_v, out_hbm.at[pl.ds(base, bpw)])

    return sc_kernel


def kernel(sources, targets, table, w, b):
    B = sources.shape[0]
    D = table.shape[1]
    src2 = sources.reshape(NW * (B // NW // CH), CH)
    tgt2 = targets.reshape(NW * (B // NW // CH), CH)
    dots = _make_sc_kernel(B, D)(src2, tgt2, table, w.reshape(1), b)
    return dots.reshape(B, 1)


# speedup vs baseline: 1.0463x; 1.0463x over previous
import functools

import jax
import jax.numpy as jnp
from jax import lax
from jax.experimental import pallas as pl
from jax.experimental.pallas import tpu as pltpu
from jax.experimental.pallas import tpu_sc as plsc

NC = 2
NS = 16
L = 16
NW = NC * NS
CH = 128


def _make_sc_kernel(B: int, D: int):
    bpw = B // NW
    nch = bpw // CH
    gpc = CH // L
    mesh = plsc.VectorSubcoreMesh(core_axis_name="c", subcore_axis_name="s")

    @functools.partial(
        pl.kernel,
        out_type=jax.ShapeDtypeStruct((B,), jnp.float32),
        mesh=mesh,
        compiler_params=pltpu.CompilerParams(
            needs_layout_passes=False, use_tc_tiling_on_sc=False),
        scratch_types=[
            pltpu.VMEM((nch, CH), jnp.int32),
            pltpu.VMEM((nch, CH), jnp.int32),
            pltpu.VMEM((nch, CH, D), jnp.float32),
            pltpu.VMEM((nch, CH, D), jnp.float32),
            pltpu.VMEM((bpw,), jnp.float32),
            pltpu.VMEM((L,), jnp.float32),
            pltpu.SemaphoreType.DMA((nch + 2,)),
        ],
    )
    def sc_kernel(src_hbm, tgt_hbm, table_hbm, w_hbm, b_hbm, out_hbm,
                  idx_s, idx_t, rows_s, rows_t, out_v, wb_sm, sems):
        wid = lax.axis_index("s") * NC + lax.axis_index("c")
        base = wid * bpw

        c_is = pltpu.async_copy(src_hbm.at[pl.ds(wid * nch, nch)], idx_s,
                                sems.at[nch])
        c_it = pltpu.async_copy(tgt_hbm.at[pl.ds(wid * nch, nch)], idx_t,
                                sems.at[nch + 1])
        c_is.wait()
        c_it.wait()
        copies = []
        for j in range(nch):
            copies.append((
                pltpu.async_copy(table_hbm.at[idx_s.at[j]], rows_s.at[j], sems.at[j]),
                pltpu.async_copy(table_hbm.at[idx_t.at[j]], rows_t.at[j], sems.at[j]),
            ))
        c_w = pltpu.async_copy(w_hbm, wb_sm.at[pl.ds(0, 1)], sems.at[nch])
        c_b = pltpu.async_copy(b_hbm, wb_sm.at[pl.ds(8, 1)], sems.at[nch + 1])
        c_w.wait()
        c_b.wait()

        wb_vec = wb_sm[...]
        w_vec = jnp.full((L,), wb_vec[0], jnp.float32)
        b_vec = jnp.full((L,), wb_vec[8], jnp.float32)
        lane = lax.iota(jnp.int32, L)

        for j in range(nch):
            cs, ct = copies[j]
            cs.wait()
            ct.wait()

            def group(g, _, j=j):
                rows = g * L + lane
                acc = jnp.zeros((L,), jnp.float32)
                diag = lane
                for d in range(D):
                    sv = plsc.load_gather(rows_s.at[j], [rows, diag])
                    tv = plsc.load_gather(rows_t.at[j], [rows, diag])
                    acc = acc + sv * tv
                    diag = (diag + 1) & (D - 1)
                out_v[pl.ds(j * CH + g * L, L)] = acc * w_vec + b_vec
                return _

            lax.fori_loop(0, gpc, group, 0)

        pltpu.sync_copy(out_v, out_hbm.at[pl.ds(base, bpw)])

    return sc_kernel


def kernel(sources, targets, table, w, b):
    B = sources.shape[0]
    D = table.shape[1]
    src2 = sources.reshape(NW * (B // NW // CH), CH)
    tgt2 = targets.reshape(NW * (B // NW // CH), CH)
    dots = _make_sc_kernel(B, D)(src2, tgt2, table, w.reshape(1), b)
    return dots.reshape(B, 1)
